# baseline (speedup 1.0000x reference)
import functools

import jax
import jax.numpy as jnp
from jax import lax
from jax.experimental import pallas as pl
from jax.experimental.pallas import tpu as pltpu
from jax.experimental.pallas import tpu_sc as plsc

NC, NS, L = 2, 16, 16
NW = NC * NS
B = 16384
V = 256
D = V * 3
T = 254
NG = (T + L - 1) // L
ROWS_PER_W = B // NW
CH = 64
NCHUNK = ROWS_PER_W // CH

INV_CBRT_MAGIC = 1420470954


def _sc_body(x_hbm, m_hbm, out_hbm, mbuf, mcol, xbuf, accbuf,
             sem0, sem1, sem2, sem3):
    wid = lax.axis_index("s") * NC + lax.axis_index("c")
    lane = lax.iota(jnp.int32, L)
    sems = (sem0, sem1)
    osems = (sem2, sem3)

    def in_slice(ci):
        return x_hbm.at[pl.ds(wid * ROWS_PER_W + ci * CH, CH), :]

    def out_slice(ci):
        return out_hbm.at[pl.ds(wid * ROWS_PER_W + ci * CH, CH), :]

    pltpu.sync_copy(m_hbm, mbuf)
    for g in range(NG):
        t = g * L + lane
        valid = t < T
        tc = jnp.where(valid, t, 0)
        for j in range(3):
            mj = plsc.load_gather(mbuf, [tc * 3 + j])
            for comp in range(3):
                col = jnp.where(valid, mj * 3 + comp, 0)
                mcol[pl.ds((g * 9 + j * 3 + comp) * L, L)] = col

    def compute_chunk(ci, buf):
        xb = xbuf.at[buf]

        GB = 4
        for gp in range(NG // GB):
            groups = [GB * gp + i for i in range(GB)]
            colsb = [[mcol[pl.ds((g * 9 + s) * L, L)] for s in range(9)]
                     for g in groups]
            gmask = (groups[-1] * L + lane) < T

            @plsc.parallel_loop(0, CH, 1)
            def tri_body(r, colsb=colsb, gp=gp, gmask=gmask):
                rvec = jnp.full((L,), r, jnp.int32)
                tot = None
                for gi in range(GB):
                    vals = [plsc.load_gather(xb, [rvec, c])
                            for c in colsb[gi]]
                    ax, ay, az, bx, by, bz, cx, cy, cz = vals
                    det = (ax * (by * cz - bz * cy)
                           - ay * (bx * cz - bz * cx)
                           + az * (bx * cy - by * cx))
                    ad = jnp.abs(det)
                    if gp == NG // GB - 1 and gi == GB - 1:
                        ad = jnp.where(gmask, ad, 0.0)
                    tot = ad if tot is None else tot + ad
                if gp == 0:
                    accbuf[pl.ds(r * L, L)] = tot
                else:
                    plsc.addupdate(accbuf.at[pl.ds(r * L, L)], tot)

        @plsc.parallel_loop(0, CH, 1)
        def norm_body(r):
            acc = accbuf[pl.ds(r * L, L)]
            vol = jnp.sum(acc) * (1.0 / 6.0)
            v = jnp.full((L,), vol, jnp.float32)
            bits = plsc.bitcast(v, jnp.int32)
            y = plsc.bitcast(INV_CBRT_MAGIC - lax.div(bits, jnp.int32(3)),
                             jnp.float32)
            for _ in range(3):
                y = y * (4.0 - v * y * y * y) * (1.0 / 3.0)
            for j in range(D // L):
                xbuf[buf, r, pl.ds(j * L, L)] = (
                    xbuf[buf, r, pl.ds(j * L, L)] * y)

        pltpu.async_copy(xb, out_slice(ci), osems[buf])

    pltpu.async_copy(in_slice(0), xbuf.at[0], sems[0])

    def pair_body(k, carry):
        for parity in range(2):
            ci = 2 * k + parity
            pltpu.make_async_copy(in_slice(ci), xbuf.at[parity],
                                  sems[parity]).wait()

            @pl.when(ci + 1 < NCHUNK)
            def _start_next(ci=ci, parity=parity):
                @pl.when(ci >= 1)
                def _drain_out(ci=ci, parity=parity):
                    pltpu.make_async_copy(xbuf.at[1 - parity],
                                          out_slice(ci - 1),
                                          osems[1 - parity]).wait()

                pltpu.async_copy(in_slice(ci + 1), xbuf.at[1 - parity],
                                 sems[1 - parity])

            compute_chunk(ci, parity)
        return carry

    lax.fori_loop(0, NCHUNK // 2, pair_body, 0)
    pltpu.make_async_copy(xbuf.at[0], out_slice(NCHUNK - 2), osems[0]).wait()
    pltpu.make_async_copy(xbuf.at[1], out_slice(NCHUNK - 1), osems[1]).wait()


_mesh = plsc.VectorSubcoreMesh(
    core_axis_name="c", subcore_axis_name="s", num_cores=NC, num_subcores=NS
)

_sc_call = functools.partial(
    pl.kernel,
    out_type=jax.ShapeDtypeStruct((B, D), jnp.float32),
    mesh=_mesh,
    scratch_types=[
        pltpu.VMEM((T * 3 + 6,), jnp.int32),
        pltpu.VMEM((NG * 9 * L,), jnp.int32),
        pltpu.VMEM((2, CH, D), jnp.float32),
        pltpu.VMEM((CH * L,), jnp.float32),
        pltpu.SemaphoreType.DMA,
        pltpu.SemaphoreType.DMA,
        pltpu.SemaphoreType.DMA,
        pltpu.SemaphoreType.DMA,
    ],
    compiler_params=pltpu.CompilerParams(
        needs_layout_passes=False, use_tc_tiling_on_sc=True
    ),
)(_sc_body)


@jax.jit
def kernel(x, M):
    m1 = jnp.concatenate([M.reshape(-1).astype(jnp.int32),
                          jnp.zeros((6,), jnp.int32)])
    return _sc_call(x, m1)


# speedup vs baseline: 1.3914x; 1.3914x over previous
import functools

import jax
import jax.numpy as jnp
from jax import lax
from jax.experimental import pallas as pl
from jax.experimental.pallas import tpu as pltpu
from jax.experimental.pallas import tpu_sc as plsc

NC, NS, L = 2, 16, 16
NW = NC * NS
B = 16384
V = 256
D = V * 3
T = 254
NG = (T + L - 1) // L
ROWS_PER_W = B // NW
CH = 64
NCHUNK = ROWS_PER_W // CH

INV_CBRT_MAGIC = 1420470954


def _sc_body(x_hbm, m_hbm, out_hbm, mbuf, mcol, xbuf, accbuf,
             sem0, sem1, sem2, sem3):
    wid = lax.axis_index("s") * NC + lax.axis_index("c")
    lane = lax.iota(jnp.int32, L)
    sems = (sem0, sem1)
    osems = (sem2, sem3)

    def in_slice(ci):
        return x_hbm.at[pl.ds(wid * ROWS_PER_W + ci * CH, CH), :]

    def out_slice(ci):
        return out_hbm.at[pl.ds(wid * ROWS_PER_W + ci * CH, CH), :]

    pltpu.sync_copy(m_hbm, mbuf)
    for g in range(NG):
        t = g * L + lane
        valid = t < T
        tc = jnp.where(valid, t, 0)
        for j in range(3):
            mj = plsc.load_gather(mbuf, [tc * 3 + j])
            for comp in range(3):
                col = jnp.where(valid, mj * 3 + comp, 0)
                mcol[pl.ds((g * 9 + j * 3 + comp) * L, L)] = col

    def compute_chunk(ci, buf):
        xb = xbuf.at[buf]

        GB = 2
        for gp in range(NG // GB):
            groups = [GB * gp + i for i in range(GB)]
            colsb = [[mcol[pl.ds((g * 9 + s) * L, L)] for s in range(9)]
                     for g in groups]
            gmask = (groups[-1] * L + lane) < T

            @plsc.parallel_loop(0, CH, 1)
            def tri_body(r, colsb=colsb, gp=gp, gmask=gmask):
                rvec = jnp.full((L,), r, jnp.int32)
                tot = None
                for gi in range(GB):
                    vals = [plsc.load_gather(xb, [rvec, c])
                            for c in colsb[gi]]
                    ax, ay, az, bx, by, bz, cx, cy, cz = vals
                    det = (ax * (by * cz - bz * cy)
                           - ay * (bx * cz - bz * cx)
                           + az * (bx * cy - by * cx))
                    ad = jnp.abs(det)
                    if gp == NG // GB - 1 and gi == GB - 1:
                        ad = jnp.where(gmask, ad, 0.0)
                    tot = ad if tot is None else tot + ad
                if gp == 0:
                    accbuf[pl.ds(r * L, L)] = tot
                else:
                    plsc.addupdate(accbuf.at[pl.ds(r * L, L)], tot)

        @plsc.parallel_loop(0, CH, 1)
        def norm_body(r):
            acc = accbuf[pl.ds(r * L, L)]
            vol = jnp.sum(acc) * (1.0 / 6.0)
            v = jnp.full((L,), vol, jnp.float32)
            bits = plsc.bitcast(v, jnp.int32)
            y = plsc.bitcast(INV_CBRT_MAGIC - lax.div(bits, jnp.int32(3)),
                             jnp.float32)
            for _ in range(3):
                y = y * (4.0 - v * y * y * y) * (1.0 / 3.0)
            for j in range(D // L):
                xbuf[buf, r, pl.ds(j * L, L)] = (
                    xbuf[buf, r, pl.ds(j * L, L)] * y)

        pltpu.async_copy(xb, out_slice(ci), osems[buf])

    pltpu.async_copy(in_slice(0), xbuf.at[0], sems[0])

    def pair_body(k, carry):
        for parity in range(2):
            ci = 2 * k + parity
            pltpu.make_async_copy(in_slice(ci), xbuf.at[parity],
                                  sems[parity]).wait()

            @pl.when(ci + 1 < NCHUNK)
            def _start_next(ci=ci, parity=parity):
                @pl.when(ci >= 1)
                def _drain_out(ci=ci, parity=parity):
                    pltpu.make_async_copy(xbuf.at[1 - parity],
                                          out_slice(ci - 1),
                                          osems[1 - parity]).wait()

                pltpu.async_copy(in_slice(ci + 1), xbuf.at[1 - parity],
                                 sems[1 - parity])

            compute_chunk(ci, parity)
        return carry

    lax.fori_loop(0, NCHUNK // 2, pair_body, 0)
    pltpu.make_async_copy(xbuf.at[0], out_slice(NCHUNK - 2), osems[0]).wait()
    pltpu.make_async_copy(xbuf.at[1], out_slice(NCHUNK - 1), osems[1]).wait()


_mesh = plsc.VectorSubcoreMesh(
    core_axis_name="c", subcore_axis_name="s", num_cores=NC, num_subcores=NS
)

_sc_call = functools.partial(
    pl.kernel,
    out_type=jax.ShapeDtypeStruct((B, D), jnp.float32),
    mesh=_mesh,
    scratch_types=[
        pltpu.VMEM((T * 3 + 6,), jnp.int32),
        pltpu.VMEM((NG * 9 * L,), jnp.int32),
        pltpu.VMEM((2, CH, D), jnp.float32),
        pltpu.VMEM((CH * L,), jnp.float32),
        pltpu.SemaphoreType.DMA,
        pltpu.SemaphoreType.DMA,
        pltpu.SemaphoreType.DMA,
        pltpu.SemaphoreType.DMA,
    ],
    compiler_params=pltpu.CompilerParams(
        needs_layout_passes=False, use_tc_tiling_on_sc=True
    ),
)(_sc_body)


@jax.jit
def kernel(x, M):
    m1 = jnp.concatenate([M.reshape(-1).astype(jnp.int32),
                          jnp.zeros((6,), jnp.int32)])
    return _sc_call(x, m1)
